# Initial kernel scaffold; baseline (speedup 1.0000x reference)
import jax
import jax.numpy as jnp
from jax.experimental import pallas as pl


def kernel(X, table, W):
    raise NotImplementedError("write your pallas kernel here")


# speedup vs baseline: 6.8992x; 6.8992x over previous
import functools

import jax
import jax.numpy as jnp
from jax.experimental import pallas as pl
from jax.experimental.pallas import tpu as pltpu
from jax.experimental.pallas import tpu_sc as plsc

_HID = 128
_GATHER_WINDOW = 128


def _project_body(table_ref, w_ref, p_ref):
    p_ref[...] = jnp.dot(table_ref[...], w_ref[...],
                         preferred_element_type=jnp.float32)


def _project(table, W):
    return pl.pallas_call(
        _project_body,
        out_shape=jax.ShapeDtypeStruct((table.shape[0], W.shape[1]),
                                       jnp.float32),
    )(table, W)


def _gather(p, idx):
    n = idx.shape[0]
    idx2 = idx.reshape(1, n)
    mesh = plsc.VectorSubcoreMesh(core_axis_name="c", subcore_axis_name="s")

    @functools.partial(
        pl.kernel,
        out_type=jax.ShapeDtypeStruct((n, _HID), jnp.float32),
        mesh=mesh,
    )
    def k(p_hbm, i_hbm, o_hbm):
        def body(i_vmem, o_vmem):
            pltpu.sync_copy(p_hbm.at[i_vmem.at[0]], o_vmem)

        pltpu.emit_pipeline(
            body,
            grid=(n // _GATHER_WINDOW,),
            in_specs=[pl.BlockSpec((1, _GATHER_WINDOW), lambda i: (0, i))],
            out_specs=[pl.BlockSpec((_GATHER_WINDOW, _HID),
                                    lambda i: (i, 0))],
            core_axis_name=("c", "s"),
            dimension_semantics=(pltpu.PARALLEL,),
        )(i_hbm, o_hbm)

    return k(p, idx2)


def kernel(X, table, W):
    p = _project(table, W)
    flat = X.reshape(-1).astype(jnp.int32)
    return _gather(p, flat)


# speedup vs baseline: 7.4191x; 1.0754x over previous
import functools

import jax
import jax.numpy as jnp
from jax.experimental import pallas as pl
from jax.experimental.pallas import tpu as pltpu
from jax.experimental.pallas import tpu_sc as plsc

_HID = 128
_GATHER_WINDOW = 128


def _project_body(table_ref, w_ref, p_ref):
    p_ref[...] = jnp.dot(table_ref[...], w_ref[...],
                         preferred_element_type=jnp.float32)


def _project(table, W):
    return pl.pallas_call(
        _project_body,
        out_shape=jax.ShapeDtypeStruct((table.shape[0], W.shape[1]),
                                       jnp.float32),
    )(table, W)


_WINDOWS_PER_STEP = 2


def _gather(p, idx):
    n = idx.shape[0]
    w, k_w = _GATHER_WINDOW, _WINDOWS_PER_STEP
    idx2 = idx.reshape(n // w, w)
    mesh = plsc.VectorSubcoreMesh(core_axis_name="c", subcore_axis_name="s")

    @functools.partial(
        pl.kernel,
        out_type=jax.ShapeDtypeStruct((n, _HID), jnp.float32),
        mesh=mesh,
        scratch_types=[pltpu.SemaphoreType.DMA] * k_w,
    )
    def k(p_hbm, i_hbm, o_hbm, *sems):
        def body(i_vmem, o_vmem):
            copies = [
                pltpu.async_copy(p_hbm.at[i_vmem.at[j]],
                                 o_vmem.at[pl.ds(j * w, w)], sems[j])
                for j in range(k_w)
            ]
            for c in copies:
                c.wait()

        pltpu.emit_pipeline(
            body,
            grid=(n // (w * k_w),),
            in_specs=[pl.BlockSpec((k_w, w), lambda i: (i, 0))],
            out_specs=[pl.BlockSpec((k_w * w, _HID), lambda i: (i, 0))],
            core_axis_name=("c", "s"),
            dimension_semantics=(pltpu.PARALLEL,),
        )(i_hbm, o_hbm)

    return k(p, idx2)


def kernel(X, table, W):
    p = _project(table, W)
    flat = X.reshape(-1).astype(jnp.int32)
    return _gather(p, flat)
